# baseline (speedup 1.0000x reference)
import functools

import jax
import jax.numpy as jnp
from jax import lax
from jax.experimental import pallas as pl
from jax.experimental.pallas import tpu as pltpu
from jax.experimental.pallas import tpu_sc as plsc

N = 10000
E = 320000
D = 128
DE = 16

NC = 2
NS = 16
NW = NC * NS
GW = 16
EPW = E // NW
CPW = EPW // GW
WRS = 640
TAIL = N - WRS * (NS - 1)

EB = 4000


_PHI_COEF = (
    1.4136380326e-01, -7.0296862832e-02, 5.1519425905e-02, -4.0424595335e-02,
    3.1452709101e-02, -2.3426441593e-02, 1.6381021741e-02, -1.0576370440e-02,
    6.3506999802e-03, -3.7438653127e-03, 1.9869238533e-03, -7.0392840761e-04,
    2.8833139909e-04, -2.9909788660e-04, 1.2820092414e-04,
)


def _gelu(x):
    xc = jnp.minimum(jnp.maximum(x, -5.0), 5.0)
    u = xc * xc * 0.08 - 1.0
    acc = jnp.full_like(x, _PHI_COEF[-1])
    for c in _PHI_COEF[-2::-1]:
        acc = acc * u + c
    phi = xc * acc + 0.5
    return x * phi


def _pre_body(h_ref, w_ref, b_ref, o_ref):
    o_ref[...] = jnp.dot(h_ref[...], w_ref[...],
                         preferred_element_type=jnp.float32) + b_ref[...]


def _q_body(ea_ref, w_ref, o_ref):
    o_ref[...] = jnp.dot(ea_ref[...], w_ref[...],
                         preferred_element_type=jnp.float32)


def _out_body(h_ref, a_ref, c_ref, w2_ref, b2_ref, gm_ref, bt_ref, o_ref):
    agg = a_ref[0] + a_ref[1]
    cnt = jnp.sum(c_ref[...], axis=0)
    m = jnp.dot(agg, w2_ref[...], preferred_element_type=jnp.float32)
    x = h_ref[...] + m + cnt[:, None] * b2_ref[...]
    mu = jnp.mean(x, axis=1, keepdims=True)
    xc = x - mu
    var = jnp.mean(xc * xc, axis=1, keepdims=True)
    o_ref[...] = xc * lax.rsqrt(var + 1e-5) * gm_ref[...] + bt_ref[...]


def _sc_counts(dst2):
    mesh = plsc.VectorSubcoreMesh(core_axis_name="c", subcore_axis_name="s")

    @functools.partial(
        pl.kernel,
        out_type=jax.ShapeDtypeStruct((NW * N,), jnp.float32),
        mesh=mesh,
        compiler_params=pltpu.CompilerParams(needs_layout_passes=False),
        scratch_types=[
            pltpu.VMEM((EPW,), jnp.int32),
            pltpu.VMEM((N,), jnp.float32),
        ],
    )
    def k(d_hbm, cnt_hbm, idx_v, cnt_v):
        c = lax.axis_index("c")
        s = lax.axis_index("s")
        wid = c * NS + s
        zero16 = jnp.zeros((16,), jnp.float32)
        one16 = jnp.full((16,), 1.0, jnp.float32)

        @pl.loop(0, N, step=16)
        def _(i):
            cnt_v[pl.ds(i, 16)] = zero16

        pltpu.sync_copy(d_hbm.at[wid], idx_v)

        @pl.loop(0, EPW, step=16)
        def _(i):
            plsc.addupdate_scatter(cnt_v, [idx_v[pl.ds(i, 16)]], one16)

        pltpu.sync_copy(cnt_v, cnt_hbm.at[pl.ds(wid * N, N)])

    return k(dst2)


def _sc_fused(P, Q, src3d, dst3d):
    mesh = plsc.VectorSubcoreMesh(core_axis_name="c", subcore_axis_name="s")

    @functools.partial(
        pl.kernel,
        out_type=jax.ShapeDtypeStruct((NC, N, D), jnp.float32),
        mesh=mesh,
        compiler_params=pltpu.CompilerParams(needs_layout_passes=False),
        scratch_types=[
            pltpu.VMEM((EPW,), jnp.int32),
            pltpu.VMEM((EPW,), jnp.int32),
            pltpu.VMEM((GW, D), jnp.float32),
            pltpu.VMEM((GW, D), jnp.float32),
            pltpu.VMEM((GW, D), jnp.float32),
            pltpu.VMEM((GW, D), jnp.float32),
            pltpu.VMEM_SHARED((N, D), jnp.float32),
            pltpu.SemaphoreType.DMA,
            pltpu.SemaphoreType.DMA,
            pltpu.SemaphoreType.DMA,
            pltpu.SemaphoreType.DMA,
            pltpu.SemaphoreType.DMA,
            pltpu.SemaphoreType.DMA,
        ],
    )
    def k(p_hbm, q_hbm, s_hbm, d_hbm, agg_hbm,
          sidx_v, didx_v, pb0, pb1, qb0, qb1, agg_sh,
          gs0, gs1, qs0, qs1, ss0, ss1):
        c = lax.axis_index("c")
        s = lax.axis_index("s")
        wid = c * NS + s
        zero16 = jnp.zeros((16,), jnp.float32)
        nzc = jnp.where(s == NS - 1, TAIL // GW, WRS // GW)

        @pl.loop(0, GW)
        def _(r):
            @pl.loop(0, D, step=16)
            def _(cc):
                qb0[r, pl.ds(cc, 16)] = zero16

        @pl.loop(0, nzc)
        def _(r):
            pltpu.sync_copy(qb0, agg_sh.at[pl.ds(s * WRS + r * GW, GW)])

        plsc.subcore_barrier()

        pltpu.sync_copy(s_hbm.at[wid], sidx_v)
        pltpu.sync_copy(d_hbm.at[wid], didx_v)

        def _gelu_chunk(pb, qb):
            @pl.loop(0, GW)
            def _(r):
                for t in range(D // 16):
                    sl = pl.ds(t * 16, 16)
                    qb[r, sl] = _gelu(pb[r, sl] + qb[r, sl])

        base = wid * EPW

        def _src16(j):
            return sidx_v[pl.ds(j * GW, GW)]

        def _dst16(j):
            return didx_v[pl.ds(j * GW, GW)]

        pltpu.async_copy(p_hbm.at[_src16(0)], pb0, gs0)
        pltpu.async_copy(q_hbm.at[pl.ds(base, GW)], qb0, qs0)
        pltpu.async_copy(p_hbm.at[_src16(1)], pb1, gs1)
        pltpu.async_copy(q_hbm.at[pl.ds(base + GW, GW)], qb1, qs1)

        @pl.loop(0, CPW - 1, step=2)
        def _(j):
            pltpu.make_async_copy(p_hbm.at[_src16(j)], pb0, gs0).wait()
            pltpu.make_async_copy(q_hbm.at[pl.ds(base + j * GW, GW)], qb0,
                                  qs0).wait()
            _gelu_chunk(pb0, qb0)
            pltpu.async_copy(p_hbm.at[_src16(j + 2)], pb0, gs0)
            s0 = pltpu.async_copy(qb0, agg_sh.at[_dst16(j)], ss0, add=True)

            pltpu.make_async_copy(p_hbm.at[_src16(j + 1)], pb1, gs1).wait()
            pltpu.make_async_copy(q_hbm.at[pl.ds(base + (j + 1) * GW, GW)],
                                  qb1, qs1).wait()
            _gelu_chunk(pb1, qb1)

            @pl.when(j + 3 < CPW)
            def _():
                pltpu.async_copy(p_hbm.at[_src16(j + 3)], pb1, gs1)

            s1 = pltpu.async_copy(qb1, agg_sh.at[_dst16(j + 1)], ss1,
                                  add=True)

            s0.wait()
            pltpu.async_copy(q_hbm.at[pl.ds(base + (j + 2) * GW, GW)],
                             qb0, qs0)
            s1.wait()

            @pl.when(j + 3 < CPW)
            def _():
                pltpu.async_copy(q_hbm.at[pl.ds(base + (j + 3) * GW, GW)],
                                 qb1, qs1)

        pltpu.make_async_copy(p_hbm.at[_src16(CPW - 1)], pb0, gs0).wait()
        pltpu.make_async_copy(q_hbm.at[pl.ds(base + (CPW - 1) * GW, GW)],
                              qb0, qs0).wait()
        _gelu_chunk(pb0, qb0)
        pltpu.sync_copy(qb0, agg_sh.at[_dst16(CPW - 1)], add=True)

        plsc.subcore_barrier()

        @pl.loop(0, nzc)
        def _(r):
            pltpu.sync_copy(agg_sh.at[pl.ds(s * WRS + r * GW, GW)], pb0)
            pltpu.sync_copy(pb0, agg_hbm.at[c, pl.ds(s * WRS + r * GW, GW)])

    return k(P, Q, src3d, dst3d)


def kernel(H, edge_index, edge_attr, W1, b1, W2, b2, gamma, beta):
    src = edge_index[0].astype(jnp.int32)
    dst = edge_index[1].astype(jnp.int32)
    W1a = W1[:D]
    W1b = W1[D:]
    b1r = b1.reshape(1, D)
    b2r = b2.reshape(1, D)
    gmr = gamma.reshape(1, D)
    btr = beta.reshape(1, D)

    P = pl.pallas_call(
        _pre_body,
        out_shape=jax.ShapeDtypeStruct((N, D), jnp.float32),
    )(H, W1a, b1r)

    Q = pl.pallas_call(
        _q_body,
        grid=(E // EB,),
        in_specs=[
            pl.BlockSpec((EB, DE), lambda i: (i, 0)),
            pl.BlockSpec((DE, D), lambda i: (0, 0)),
        ],
        out_specs=pl.BlockSpec((EB, D), lambda i: (i, 0)),
        out_shape=jax.ShapeDtypeStruct((E, D), jnp.float32),
    )(edge_attr, W1b)

    counts = _sc_counts(dst.reshape(NW, EPW)).reshape(NW, N)
    aggP = _sc_fused(P, Q, src.reshape(NW, EPW), dst.reshape(NW, EPW))

    out = pl.pallas_call(
        _out_body,
        out_shape=jax.ShapeDtypeStruct((N, D), jnp.float32),
    )(H, aggP, counts, W2, b2r, gmr, btr)

    return out


# speedup vs baseline: 1.2161x; 1.2161x over previous
import functools

import jax
import jax.numpy as jnp
from jax import lax
from jax.experimental import pallas as pl
from jax.experimental.pallas import tpu as pltpu
from jax.experimental.pallas import tpu_sc as plsc

N = 10000
E = 320000
D = 128
DE = 16

NC = 2
NS = 16
NW = NC * NS
GW = 16
EPW = E // NW
CPW = EPW // GW
WRS = 640
TAIL = N - WRS * (NS - 1)

EB = 4000


_PHI_COEF = (
    1.4136384155e-01, -7.0296096625e-02, 5.1516091368e-02, -4.0447051858e-02,
    3.1500128901e-02, -2.3239361995e-02, 1.6128605679e-02, -1.1241586982e-02,
    6.9880958648e-03, -2.5978223607e-03, 1.1655016453e-03, -1.6490621540e-03,
    8.0896297553e-04,
)


def _gelu_multi(xs):
    c = _PHI_COEF
    xcs = [jnp.minimum(jnp.maximum(x, -5.0), 5.0) for x in xs]
    us = [xc * xc * 0.08 - 1.0 for xc in xcs]
    u2s = [u * u for u in us]
    u4s = [u2 * u2 for u2 in u2s]
    u8s = [u4 * u4 for u4 in u4s]
    b0s = [c[1] * u + c[0] for u in us]
    b1s = [c[3] * u + c[2] for u in us]
    b2s = [c[5] * u + c[4] for u in us]
    b3s = [c[7] * u + c[6] for u in us]
    b4s = [c[9] * u + c[8] for u in us]
    b5s = [c[11] * u + c[10] for u in us]
    d0s = [b1 * u2 + b0 for b1, u2, b0 in zip(b1s, u2s, b0s)]
    d1s = [b3 * u2 + b2 for b3, u2, b2 in zip(b3s, u2s, b2s)]
    d2s = [b5 * u2 + b4 for b5, u2, b4 in zip(b5s, u2s, b4s)]
    d3s = [c[12] * u4 + d2 for u4, d2 in zip(u4s, d2s)]
    e0s = [d1 * u4 + d0 for d1, u4, d0 in zip(d1s, u4s, d0s)]
    ps = [d3 * u8 + e0 for d3, u8, e0 in zip(d3s, u8s, e0s)]
    phis = [xc * p + 0.5 for xc, p in zip(xcs, ps)]
    return [x * phi for x, phi in zip(xs, phis)]


def _pre_body(h_ref, w_ref, b_ref, o_ref):
    o_ref[...] = jnp.dot(h_ref[...], w_ref[...],
                         preferred_element_type=jnp.float32) + b_ref[...]


def _q_body(ea_ref, w_ref, o_ref):
    o_ref[...] = jnp.dot(ea_ref[...], w_ref[...],
                         preferred_element_type=jnp.float32)


def _out_body(h_ref, a_ref, c_ref, w2_ref, b2_ref, gm_ref, bt_ref, o_ref):
    agg = a_ref[0] + a_ref[1]
    cnt = jnp.sum(c_ref[...], axis=0)
    m = jnp.dot(agg, w2_ref[...], preferred_element_type=jnp.float32)
    x = h_ref[...] + m + cnt[:, None] * b2_ref[...]
    mu = jnp.mean(x, axis=1, keepdims=True)
    xc = x - mu
    var = jnp.mean(xc * xc, axis=1, keepdims=True)
    o_ref[...] = xc * lax.rsqrt(var + 1e-5) * gm_ref[...] + bt_ref[...]


def _sc_counts(dst2):
    mesh = plsc.VectorSubcoreMesh(core_axis_name="c", subcore_axis_name="s")

    @functools.partial(
        pl.kernel,
        out_type=jax.ShapeDtypeStruct((NW * N,), jnp.float32),
        mesh=mesh,
        compiler_params=pltpu.CompilerParams(needs_layout_passes=False),
        scratch_types=[
            pltpu.VMEM((EPW,), jnp.int32),
            pltpu.VMEM((N,), jnp.float32),
        ],
    )
    def k(d_hbm, cnt_hbm, idx_v, cnt_v):
        c = lax.axis_index("c")
        s = lax.axis_index("s")
        wid = c * NS + s
        zero16 = jnp.zeros((16,), jnp.float32)
        one16 = jnp.full((16,), 1.0, jnp.float32)

        @pl.loop(0, N, step=16)
        def _(i):
            cnt_v[pl.ds(i, 16)] = zero16

        pltpu.sync_copy(d_hbm.at[wid], idx_v)

        @pl.loop(0, EPW, step=16)
        def _(i):
            plsc.addupdate_scatter(cnt_v, [idx_v[pl.ds(i, 16)]], one16)

        pltpu.sync_copy(cnt_v, cnt_hbm.at[pl.ds(wid * N, N)])

    return k(dst2)


def _sc_fused(P, Q, src3d, dst3d):
    mesh = plsc.VectorSubcoreMesh(core_axis_name="c", subcore_axis_name="s")

    @functools.partial(
        pl.kernel,
        out_type=jax.ShapeDtypeStruct((NC, N, D), jnp.float32),
        mesh=mesh,
        compiler_params=pltpu.CompilerParams(needs_layout_passes=False),
        scratch_types=[
            pltpu.VMEM((EPW,), jnp.int32),
            pltpu.VMEM((EPW,), jnp.int32),
            pltpu.VMEM((GW, D), jnp.float32),
            pltpu.VMEM((GW, D), jnp.float32),
            pltpu.VMEM((GW, D), jnp.float32),
            pltpu.VMEM((GW, D), jnp.float32),
            pltpu.VMEM_SHARED((N, D), jnp.float32),
            pltpu.SemaphoreType.DMA,
            pltpu.SemaphoreType.DMA,
            pltpu.SemaphoreType.DMA,
            pltpu.SemaphoreType.DMA,
            pltpu.SemaphoreType.DMA,
            pltpu.SemaphoreType.DMA,
        ],
    )
    def k(p_hbm, q_hbm, s_hbm, d_hbm, agg_hbm,
          sidx_v, didx_v, pb0, pb1, qb0, qb1, agg_sh,
          gs0, gs1, qs0, qs1, ss0, ss1):
        c = lax.axis_index("c")
        s = lax.axis_index("s")
        wid = c * NS + s
        zero16 = jnp.zeros((16,), jnp.float32)
        nzc = jnp.where(s == NS - 1, TAIL // GW, WRS // GW)

        @pl.loop(0, GW)
        def _(r):
            @pl.loop(0, D, step=16)
            def _(cc):
                qb0[r, pl.ds(cc, 16)] = zero16

        @pl.loop(0, nzc)
        def _(r):
            pltpu.sync_copy(qb0, agg_sh.at[pl.ds(s * WRS + r * GW, GW)])

        plsc.subcore_barrier()

        pltpu.sync_copy(s_hbm.at[wid], sidx_v)
        pltpu.sync_copy(d_hbm.at[wid], didx_v)

        def _gelu_chunk(pb, qb):
            @pl.loop(0, GW)
            def _(r):
                sls = [pl.ds(t * 16, 16) for t in range(D // 16)]
                xs = [pb[r, sl] + qb[r, sl] for sl in sls]
                ys = _gelu_multi(xs)
                for sl, y in zip(sls, ys):
                    qb[r, sl] = y

        base = wid * EPW

        def _src16(j):
            return sidx_v[pl.ds(j * GW, GW)]

        def _dst16(j):
            return didx_v[pl.ds(j * GW, GW)]

        pltpu.async_copy(p_hbm.at[_src16(0)], pb0, gs0)
        pltpu.async_copy(q_hbm.at[pl.ds(base, GW)], qb0, qs0)
        pltpu.async_copy(p_hbm.at[_src16(1)], pb1, gs1)
        pltpu.async_copy(q_hbm.at[pl.ds(base + GW, GW)], qb1, qs1)

        @pl.loop(0, CPW - 1, step=2)
        def _(j):
            pltpu.make_async_copy(p_hbm.at[_src16(j)], pb0, gs0).wait()
            pltpu.make_async_copy(q_hbm.at[pl.ds(base + j * GW, GW)], qb0,
                                  qs0).wait()
            _gelu_chunk(pb0, qb0)
            pltpu.async_copy(p_hbm.at[_src16(j + 2)], pb0, gs0)
            s0 = pltpu.async_copy(qb0, agg_sh.at[_dst16(j)], ss0, add=True)

            pltpu.make_async_copy(p_hbm.at[_src16(j + 1)], pb1, gs1).wait()
            pltpu.make_async_copy(q_hbm.at[pl.ds(base + (j + 1) * GW, GW)],
                                  qb1, qs1).wait()
            _gelu_chunk(pb1, qb1)

            @pl.when(j + 3 < CPW)
            def _():
                pltpu.async_copy(p_hbm.at[_src16(j + 3)], pb1, gs1)

            s1 = pltpu.async_copy(qb1, agg_sh.at[_dst16(j + 1)], ss1,
                                  add=True)

            s0.wait()
            pltpu.async_copy(q_hbm.at[pl.ds(base + (j + 2) * GW, GW)],
                             qb0, qs0)
            s1.wait()

            @pl.when(j + 3 < CPW)
            def _():
                pltpu.async_copy(q_hbm.at[pl.ds(base + (j + 3) * GW, GW)],
                                 qb1, qs1)

        pltpu.make_async_copy(p_hbm.at[_src16(CPW - 1)], pb0, gs0).wait()
        pltpu.make_async_copy(q_hbm.at[pl.ds(base + (CPW - 1) * GW, GW)],
                              qb0, qs0).wait()
        _gelu_chunk(pb0, qb0)
        pltpu.sync_copy(qb0, agg_sh.at[_dst16(CPW - 1)], add=True)

        plsc.subcore_barrier()

        @pl.loop(0, nzc)
        def _(r):
            pltpu.sync_copy(agg_sh.at[pl.ds(s * WRS + r * GW, GW)], pb0)
            pltpu.sync_copy(pb0, agg_hbm.at[c, pl.ds(s * WRS + r * GW, GW)])

    return k(P, Q, src3d, dst3d)


def kernel(H, edge_index, edge_attr, W1, b1, W2, b2, gamma, beta):
    src = edge_index[0].astype(jnp.int32)
    dst = edge_index[1].astype(jnp.int32)
    W1a = W1[:D]
    W1b = W1[D:]
    b1r = b1.reshape(1, D)
    b2r = b2.reshape(1, D)
    gmr = gamma.reshape(1, D)
    btr = beta.reshape(1, D)

    P = pl.pallas_call(
        _pre_body,
        out_shape=jax.ShapeDtypeStruct((N, D), jnp.float32),
    )(H, W1a, b1r)

    Q = pl.pallas_call(
        _q_body,
        grid=(E // EB,),
        in_specs=[
            pl.BlockSpec((EB, DE), lambda i: (i, 0)),
            pl.BlockSpec((DE, D), lambda i: (0, 0)),
        ],
        out_specs=pl.BlockSpec((EB, D), lambda i: (i, 0)),
        out_shape=jax.ShapeDtypeStruct((E, D), jnp.float32),
    )(edge_attr, W1b)

    counts = _sc_counts(dst.reshape(NW, EPW)).reshape(NW, N)
    aggP = _sc_fused(P, Q, src.reshape(NW, EPW), dst.reshape(NW, EPW))

    out = pl.pallas_call(
        _out_body,
        out_shape=jax.ShapeDtypeStruct((N, D), jnp.float32),
    )(H, aggP, counts, W2, b2r, gmr, btr)

    return out


# speedup vs baseline: 2.3125x; 1.9016x over previous
import functools

import jax
import jax.numpy as jnp
from jax import lax
from jax.experimental import pallas as pl
from jax.experimental.pallas import tpu as pltpu
from jax.experimental.pallas import tpu_sc as plsc

N = 10000
E = 320000
D = 128
DE = 16

NC = 2
NS = 16
NW = NC * NS

K = 2
EK = E // K
GWIN = 128
GW = 40
EPW = EK // NW
CPW = EPW // GW
WRS = 640
TAIL = N - WRS * (NS - 1)

EB = 4000


def _erf(x):
    a1, a2, a3, a4, a5 = 0.254829592, -0.284496736, 1.421413741, -1.453152027, 1.061405429
    p = 0.3275911
    s = jnp.sign(x)
    ax = jnp.abs(x)
    t = 1.0 / (1.0 + p * ax)
    poly = ((((a5 * t + a4) * t + a3) * t + a2) * t + a1) * t
    return s * (1.0 - poly * jnp.exp(-ax * ax))


def _gelu(x):
    return 0.5 * x * (1.0 + _erf(x * 0.7071067811865476))


def _pre_body(h_ref, w_ref, b_ref, o_ref):
    o_ref[...] = jnp.dot(h_ref[...], w_ref[...],
                         preferred_element_type=jnp.float32) + b_ref[...]


def _msg_body(g_ref, ea_ref, w_ref, o_ref):
    x = g_ref[...] + jnp.dot(ea_ref[...], w_ref[...],
                             preferred_element_type=jnp.float32)
    o_ref[...] = _gelu(x)


def _out_body(h_ref, a_ref, c_ref, w2_ref, b2_ref, gm_ref, bt_ref, o_ref):
    agg = jnp.sum(a_ref[...], axis=0)
    cnt = jnp.sum(c_ref[...], axis=0)
    m = jnp.dot(agg, w2_ref[...], preferred_element_type=jnp.float32)
    x = h_ref[...] + m + cnt[:, None] * b2_ref[...]
    mu = jnp.mean(x, axis=1, keepdims=True)
    xc = x - mu
    var = jnp.mean(xc * xc, axis=1, keepdims=True)
    o_ref[...] = xc * lax.rsqrt(var + 1e-5) * gm_ref[...] + bt_ref[...]


def _sc_counts(dst2):
    mesh = plsc.VectorSubcoreMesh(core_axis_name="c", subcore_axis_name="s")
    epw = E // NW

    @functools.partial(
        pl.kernel,
        out_type=jax.ShapeDtypeStruct((NW * N,), jnp.float32),
        mesh=mesh,
        compiler_params=pltpu.CompilerParams(needs_layout_passes=False),
        scratch_types=[
            pltpu.VMEM((epw,), jnp.int32),
            pltpu.VMEM((N,), jnp.float32),
        ],
    )
    def k(d_hbm, cnt_hbm, idx_v, cnt_v):
        c = lax.axis_index("c")
        s = lax.axis_index("s")
        wid = c * NS + s
        zero16 = jnp.zeros((16,), jnp.float32)
        one16 = jnp.full((16,), 1.0, jnp.float32)

        @pl.loop(0, N, step=16)
        def _(i):
            cnt_v[pl.ds(i, 16)] = zero16

        pltpu.sync_copy(d_hbm.at[wid], idx_v)

        @pl.loop(0, epw, step=16)
        def _(i):
            plsc.addupdate_scatter(cnt_v, [idx_v[pl.ds(i, 16)]], one16)

        pltpu.sync_copy(cnt_v, cnt_hbm.at[pl.ds(wid * N, N)])

    return k(dst2)


def _sc_gather(P, src2d):
    mesh = plsc.VectorSubcoreMesh(core_axis_name="c", subcore_axis_name="s")

    @functools.partial(
        pl.kernel,
        out_type=jax.ShapeDtypeStruct((EK, D), jnp.float32),
        mesh=mesh,
    )
    def k(p_hbm, i_hbm, o_hbm):
        def body(i_vmem, o_vmem):
            pltpu.sync_copy(p_hbm.at[i_vmem.at[0]], o_vmem)

        pltpu.emit_pipeline(
            body,
            grid=(EK // GWIN,),
            in_specs=[pl.BlockSpec((1, GWIN), lambda i: (i, 0))],
            out_specs=[pl.BlockSpec((GWIN, D), lambda i: (i, 0))],
            core_axis_name=("c", "s"),
            dimension_semantics=(pltpu.PARALLEL,),
        )(i_hbm, o_hbm)

    return k(P, src2d)


def _sc_scatter(h, dst3d):
    mesh = plsc.VectorSubcoreMesh(core_axis_name="c", subcore_axis_name="s")

    @functools.partial(
        pl.kernel,
        out_type=jax.ShapeDtypeStruct((NC, N, D), jnp.float32),
        mesh=mesh,
        scratch_types=[
            pltpu.VMEM((CPW, GW), jnp.int32),
            pltpu.VMEM((GW, D), jnp.float32),
            pltpu.VMEM((GW, D), jnp.float32),
            pltpu.VMEM_SHARED((N, D), jnp.float32),
            pltpu.SemaphoreType.DMA,
            pltpu.SemaphoreType.DMA,
            pltpu.SemaphoreType.DMA,
            pltpu.SemaphoreType.DMA,
        ],
    )
    def k(h_hbm, d_hbm, agg_hbm, idx_v, rows_v, rows_v1, agg_sh,
          lsem0, lsem1, ssem0, ssem1):
        c = lax.axis_index("c")
        s = lax.axis_index("s")
        wid = c * NS + s
        zero16 = jnp.zeros((16,), jnp.float32)
        nchunk = jnp.where(s == NS - 1, TAIL // GW, WRS // GW)

        @pl.loop(0, GW)
        def _(r):
            @pl.loop(0, D, step=16)
            def _(cc):
                rows_v[r, pl.ds(cc, 16)] = zero16

        @pl.loop(0, nchunk)
        def _(r):
            pltpu.sync_copy(rows_v, agg_sh.at[pl.ds(s * WRS + r * GW, GW)])

        plsc.subcore_barrier()

        pltpu.sync_copy(d_hbm.at[wid], idx_v)

        base = wid * EPW
        pltpu.async_copy(h_hbm.at[pl.ds(base, GW)], rows_v, lsem0)
        pltpu.async_copy(h_hbm.at[pl.ds(base + GW, GW)], rows_v1, lsem1)

        @pl.loop(0, CPW - 1, step=2)
        def _(j):
            pltpu.make_async_copy(h_hbm.at[pl.ds(base + j * GW, GW)],
                                  rows_v, lsem0).wait()
            s0 = pltpu.async_copy(rows_v, agg_sh.at[idx_v.at[j]], ssem0,
                                  add=True)
            pltpu.make_async_copy(h_hbm.at[pl.ds(base + (j + 1) * GW, GW)],
                                  rows_v1, lsem1).wait()
            s1 = pltpu.async_copy(rows_v1, agg_sh.at[idx_v.at[j + 1]], ssem1,
                                  add=True)
            s0.wait()
            pltpu.async_copy(h_hbm.at[pl.ds(base + (j + 2) * GW, GW)],
                             rows_v, lsem0)
            s1.wait()

            @pl.when(j + 3 < CPW)
            def _():
                pltpu.async_copy(h_hbm.at[pl.ds(base + (j + 3) * GW, GW)],
                                 rows_v1, lsem1)

        pltpu.make_async_copy(h_hbm.at[pl.ds(base + (CPW - 1) * GW, GW)],
                              rows_v, lsem0).wait()
        pltpu.sync_copy(rows_v, agg_sh.at[idx_v.at[CPW - 1]], add=True)

        plsc.subcore_barrier()

        @pl.loop(0, nchunk)
        def _(r):
            pltpu.sync_copy(agg_sh.at[pl.ds(s * WRS + r * GW, GW)], rows_v)
            pltpu.sync_copy(rows_v, agg_hbm.at[c, pl.ds(s * WRS + r * GW, GW)])

    return k(h, dst3d)


def kernel(H, edge_index, edge_attr, W1, b1, W2, b2, gamma, beta):
    src = edge_index[0].astype(jnp.int32)
    dst = edge_index[1].astype(jnp.int32)
    W1a = W1[:D]
    W1b = W1[D:]
    b1r = b1.reshape(1, D)
    b2r = b2.reshape(1, D)
    gmr = gamma.reshape(1, D)
    btr = beta.reshape(1, D)

    P = pl.pallas_call(
        _pre_body,
        out_shape=jax.ShapeDtypeStruct((N, D), jnp.float32),
    )(H, W1a, b1r)

    counts = _sc_counts(dst.reshape(NW, E // NW)).reshape(NW, N)

    aggs = []
    for kk in range(K):
        src_k = lax.dynamic_slice_in_dim(src, kk * EK, EK)
        dst_k = lax.dynamic_slice_in_dim(dst, kk * EK, EK)
        ea_k = lax.dynamic_slice_in_dim(edge_attr, kk * EK, EK)

        G = _sc_gather(P, src_k.reshape(EK // GWIN, GWIN))

        h = pl.pallas_call(
            _msg_body,
            grid=(EK // EB,),
            in_specs=[
                pl.BlockSpec((EB, D), lambda i: (i, 0)),
                pl.BlockSpec((EB, DE), lambda i: (i, 0)),
                pl.BlockSpec((DE, D), lambda i: (0, 0)),
            ],
            out_specs=pl.BlockSpec((EB, D), lambda i: (i, 0)),
            out_shape=jax.ShapeDtypeStruct((EK, D), jnp.float32),
        )(G, ea_k, W1b)

        aggs.append(_sc_scatter(h, dst_k.reshape(NW, CPW, GW)))

    aggP = jnp.concatenate(aggs, axis=0)

    out = pl.pallas_call(
        _out_body,
        out_shape=jax.ShapeDtypeStruct((N, D), jnp.float32),
    )(H, aggP, counts, W2, b2r, gmr, btr)

    return out
